# baseline (speedup 1.0000x reference)
import functools

import jax
import jax.numpy as jnp
from jax import lax
from jax.experimental import pallas as pl
from jax.experimental.pallas import tpu as pltpu
from jax.experimental.pallas import tpu_sc as plsc

_B = 81920
_D = 1000
_NC = 2
_NS = 16
_NW = _NC * _NS
_BPW = _B // _NW
_CH = 64
_NCHUNK = _BPW // _CH


def _gather_body(idx_hbm, table_hbm, out_hbm, idx_v, rows_v, gsem):
    wid = lax.axis_index("s") * _NC + lax.axis_index("c")
    base = wid * _BPW
    pltpu.sync_copy(idx_hbm.at[wid], idx_v)

    def chunk(g, carry):
        pltpu.async_copy(table_hbm.at[idx_v.at[g]], rows_v, gsem).wait()
        pltpu.sync_copy(rows_v, out_hbm.at[pl.ds(base + g * _CH, _CH)])
        return carry

    lax.fori_loop(0, _NCHUNK, chunk, 0)


_mesh = plsc.VectorSubcoreMesh(core_axis_name="c", subcore_axis_name="s")

_gather_call = functools.partial(
    pl.kernel,
    mesh=_mesh,
    compiler_params=pltpu.CompilerParams(use_tc_tiling_on_sc=False),
    out_type=jax.ShapeDtypeStruct((_B, _D), jnp.float32),
    scratch_types=[
        pltpu.VMEM((_NCHUNK, _CH), jnp.int32),
        pltpu.VMEM((_CH, _D), jnp.float32),
        pltpu.SemaphoreType.DMA,
    ],
)(_gather_body)


@jax.jit
def kernel(idx, embedding):
    idx3 = idx.reshape(_NW, _NCHUNK, _CH).astype(jnp.int32)
    return _gather_call(idx3, embedding)


# speedup vs baseline: 1.0244x; 1.0244x over previous
import functools

import jax
import jax.numpy as jnp
from jax import lax
from jax.experimental import pallas as pl
from jax.experimental.pallas import tpu as pltpu
from jax.experimental.pallas import tpu_sc as plsc

_B = 81920
_D = 1000
_NC = 2
_NS = 16
_NW = _NC * _NS
_BPW = _B // _NW
_CH = 64
_NCHUNK = _BPW // _CH


def _gather_body(idx_hbm, table_hbm, out_hbm, idx_v, rows0, rows1,
                 gsem0, gsem1, wsem0, wsem1):
    wid = lax.axis_index("s") * _NC + lax.axis_index("c")
    base = wid * _BPW
    pltpu.sync_copy(idx_hbm.at[wid], idx_v)

    bufs = (rows0, rows1)
    gsems = (gsem0, gsem1)
    wsems = (wsem0, wsem1)

    def gather(g, b):
        pltpu.async_copy(table_hbm.at[idx_v.at[g]], bufs[b], gsems[b])

    def wait_gather(g, b):
        pltpu.make_async_copy(table_hbm.at[idx_v.at[g]], bufs[b], gsems[b]).wait()

    def write(g, b):
        pltpu.async_copy(bufs[b], out_hbm.at[pl.ds(base + g * _CH, _CH)], wsems[b])

    def wait_write(g, b):
        pltpu.make_async_copy(bufs[b], out_hbm.at[pl.ds(base + g * _CH, _CH)], wsems[b]).wait()

    gather(0, 0)
    gather(1, 1)

    def step(i, carry):
        g2 = 2 * i
        for b in (0, 1):
            g = g2 + b
            wait_gather(g, b)
            write(g, b)
            wait_write(g, b)
            gather(g + 2, b)
        return carry

    lax.fori_loop(0, (_NCHUNK - 2) // 2, step, 0)

    for b in (0, 1):
        g = _NCHUNK - 2 + b
        wait_gather(g, b)
        write(g, b)
    for b in (0, 1):
        wait_write(_NCHUNK - 2 + b, b)


_mesh = plsc.VectorSubcoreMesh(core_axis_name="c", subcore_axis_name="s")

_gather_call = functools.partial(
    pl.kernel,
    mesh=_mesh,
    compiler_params=pltpu.CompilerParams(use_tc_tiling_on_sc=False),
    out_type=jax.ShapeDtypeStruct((_B, _D), jnp.float32),
    scratch_types=[
        pltpu.VMEM((_NCHUNK, _CH), jnp.int32),
        pltpu.VMEM((_CH, _D), jnp.float32),
        pltpu.VMEM((_CH, _D), jnp.float32),
        pltpu.SemaphoreType.DMA,
        pltpu.SemaphoreType.DMA,
        pltpu.SemaphoreType.DMA,
        pltpu.SemaphoreType.DMA,
    ],
)(_gather_body)


@jax.jit
def kernel(idx, embedding):
    idx3 = idx.reshape(_NW, _NCHUNK, _CH).astype(jnp.int32)
    return _gather_call(idx3, embedding)


# speedup vs baseline: 1.0631x; 1.0377x over previous
import functools

import jax
import jax.numpy as jnp
from jax import lax
from jax.experimental import layout as jlayout
from jax.experimental import pallas as pl
from jax.experimental.pallas import tpu as pltpu
from jax.experimental.pallas import tpu_sc as plsc

_B = 81920
_D = 1000
_V = 1000
_NC = 2
_NS = 16
_NW = _NC * _NS
_BPW = _B // _NW
_CH = 64
_NCHUNK = _BPW // _CH


def _gather_body(idx_hbm, table_hbm, out_hbm, idx_v, rows_v, table_sh, gsem):
    sid = lax.axis_index("s")
    wid = sid * _NC + lax.axis_index("c")
    base = wid * _BPW
    pltpu.sync_copy(idx_hbm.at[wid], idx_v)

    s8 = sid % 8
    pltpu.sync_copy(table_hbm.at[pl.ds(s8 * 125, 125)],
                    table_sh.at[pl.ds(s8 * 125, 125)])
    plsc.subcore_barrier()

    def chunk(g, carry):
        pltpu.async_copy(table_sh.at[idx_v.at[g]], rows_v, gsem).wait()
        pltpu.sync_copy(rows_v, out_hbm.at[pl.ds(base + g * _CH, _CH)])
        return carry

    lax.fori_loop(0, _NCHUNK, chunk, 0)


_mesh = plsc.VectorSubcoreMesh(core_axis_name="c", subcore_axis_name="s")

_gather_call = functools.partial(
    pl.kernel,
    mesh=_mesh,
    compiler_params=pltpu.CompilerParams(use_tc_tiling_on_sc=False),
    out_type=jax.ShapeDtypeStruct((_B, _D), jnp.float32),
    scratch_types=[
        pltpu.VMEM((_NCHUNK, _CH), jnp.int32),
        pltpu.VMEM((_CH, _D), jnp.float32),
        pltpu.VMEM_SHARED((_V, _D), jnp.float32),
        pltpu.SemaphoreType.DMA,
    ],
)(_gather_body)

_LIN2D = jlayout.Layout(major_to_minor=(0, 1), tiling=())


def _run(idx, embedding):
    idx3 = idx.reshape(_NW, _NCHUNK, _CH).astype(jnp.int32)
    return _gather_call(idx3, embedding)


_compiled = None


def kernel(idx, embedding):
    if isinstance(idx, jax.core.Tracer):
        return _run(idx, embedding)
    global _compiled
    if _compiled is None:
        mesh = jax.sharding.get_mesh()
        if mesh is not None and not mesh.empty:
            sharding = jax.sharding.NamedSharding(
                mesh, jax.sharding.PartitionSpec())
        else:
            sharding = jax.sharding.SingleDeviceSharding(jax.devices()[0])
        fmt = jlayout.Format(_LIN2D, sharding)
        _compiled = jax.jit(_run, out_shardings=fmt)
    return _compiled(idx, embedding)


# speedup vs baseline: 1.5082x; 1.4187x over previous
import functools

import jax
import jax.numpy as jnp
from jax import lax
from jax.experimental import pallas as pl
from jax.experimental.pallas import tpu as pltpu
from jax.experimental.pallas import tpu_sc as plsc

_B = 81920
_D = 1000
_DP = 1024
_NC = 2
_NS = 16
_NW = _NC * _NS
_BPW = _B // _NW
_CH = 64
_NCHUNK = _BPW // _CH


def _gather_body(idx_hbm, tab_hbm, out_hbm, idx_v, buf, tail, gsem):
    sid = lax.axis_index("s")
    wid = sid * _NC + lax.axis_index("c")
    base = wid * _BPW
    pltpu.sync_copy(idx_hbm.at[pl.ds(base, _BPW)], idx_v)

    def chunk(g, carry):
        pltpu.async_copy(tab_hbm.at[idx_v.at[pl.ds(g * _CH, _CH)]],
                         buf, gsem).wait()
        o = pl.multiple_of(base + g * _CH, 8)
        pltpu.async_copy(buf.at[:, pl.ds(0, 896)],
                         out_hbm.at[pl.ds(o, _CH), pl.ds(0, 896)], gsem)
        for r in range(_CH):
            for k in range(6):
                tail[r, pl.ds(16 * k, 16)] = buf[r, pl.ds(896 + 16 * k, 16)]
            tail[r, pl.ds(88, 16)] = buf[r, pl.ds(984, 16)]
        pltpu.sync_copy(tail, out_hbm.at[pl.ds(o, _CH), pl.ds(896, 104)])
        pltpu.make_async_copy(buf.at[:, pl.ds(0, 896)],
                              out_hbm.at[pl.ds(o, _CH), pl.ds(0, 896)],
                              gsem).wait()
        return carry

    lax.fori_loop(0, _NCHUNK, chunk, 0)


_mesh = plsc.VectorSubcoreMesh(core_axis_name="c", subcore_axis_name="s")

_gather_call = functools.partial(
    pl.kernel,
    mesh=_mesh,
    out_type=jax.ShapeDtypeStruct((_B, _D), jnp.float32),
    scratch_types=[
        pltpu.VMEM((_BPW,), jnp.int32),
        pltpu.VMEM((_CH, _DP), jnp.float32),
        pltpu.VMEM((_CH, 104), jnp.float32),
        pltpu.SemaphoreType.DMA,
    ],
)(_gather_body)


def kernel(idx, embedding):
    idx1 = idx.reshape(-1).astype(jnp.int32)
    tabp = jnp.pad(embedding, ((0, 0), (0, _DP - _D)))
    return _gather_call(idx1, tabp)


# speedup vs baseline: 1.5295x; 1.0141x over previous
import functools

import jax
import jax.numpy as jnp
from jax import lax
from jax.experimental import pallas as pl
from jax.experimental.pallas import tpu as pltpu
from jax.experimental.pallas import tpu_sc as plsc

_B = 81920
_D = 1000
_DP = 1024
_NC = 2
_NS = 16
_NW = _NC * _NS
_BPW = _B // _NW
_CH = 40
_NCHUNK = _BPW // _CH


def _gather_body(idx_hbm, tab_hbm, out_hbm, idx_v,
                 buf0, buf1, tail0, tail1,
                 gsem0, gsem1, wa0, wa1, wb0, wb1):
    sid = lax.axis_index("s")
    wid = sid * _NC + lax.axis_index("c")
    base = wid * _BPW
    pltpu.sync_copy(idx_hbm.at[pl.ds(base, _BPW)], idx_v)

    bufs = (buf0, buf1)
    tails = (tail0, tail1)
    gsems = (gsem0, gsem1)
    was = (wa0, wa1)
    wbs = (wb0, wb1)

    def gather(g, b):
        pltpu.async_copy(tab_hbm.at[idx_v.at[pl.ds(g * _CH, _CH)]],
                         bufs[b], gsems[b])

    def wait_gather(g, b):
        pltpu.make_async_copy(tab_hbm.at[idx_v.at[pl.ds(g * _CH, _CH)]],
                              bufs[b], gsems[b]).wait()

    def emit_writes(g, b):
        o = pl.multiple_of(base + g * _CH, 8)
        buf, tail = bufs[b], tails[b]
        pltpu.async_copy(buf.at[:, pl.ds(0, 896)],
                         out_hbm.at[pl.ds(o, _CH), pl.ds(0, 896)], was[b])
        for r in range(_CH):
            for k in range(6):
                tail[r, pl.ds(16 * k, 16)] = buf[r, pl.ds(896 + 16 * k, 16)]
            tail[r, pl.ds(88, 16)] = buf[r, pl.ds(984, 16)]
        pltpu.async_copy(tail, out_hbm.at[pl.ds(o, _CH), pl.ds(896, 104)],
                         wbs[b])

    def wait_writes(g, b):
        o = pl.multiple_of(base + g * _CH, 8)
        pltpu.make_async_copy(bufs[b].at[:, pl.ds(0, 896)],
                              out_hbm.at[pl.ds(o, _CH), pl.ds(0, 896)],
                              was[b]).wait()
        pltpu.make_async_copy(tails[b],
                              out_hbm.at[pl.ds(o, _CH), pl.ds(896, 104)],
                              wbs[b]).wait()

    gather(0, 0)
    gather(1, 1)

    def step(i, carry):
        g2 = 2 * i
        for b in (0, 1):
            g = g2 + b
            wait_gather(g, b)
            emit_writes(g, b)
            wait_writes(g, b)
            gather(g + 2, b)
        return carry

    lax.fori_loop(0, (_NCHUNK - 2) // 2, step, 0)

    for b in (0, 1):
        g = _NCHUNK - 2 + b
        wait_gather(g, b)
        emit_writes(g, b)
    for b in (0, 1):
        wait_writes(_NCHUNK - 2 + b, b)


_mesh = plsc.VectorSubcoreMesh(core_axis_name="c", subcore_axis_name="s")

_gather_call = functools.partial(
    pl.kernel,
    mesh=_mesh,
    out_type=jax.ShapeDtypeStruct((_B, _D), jnp.float32),
    scratch_types=[
        pltpu.VMEM((_BPW,), jnp.int32),
        pltpu.VMEM((_CH, _DP), jnp.float32),
        pltpu.VMEM((_CH, _DP), jnp.float32),
        pltpu.VMEM((_CH, 104), jnp.float32),
        pltpu.VMEM((_CH, 104), jnp.float32),
        pltpu.SemaphoreType.DMA,
        pltpu.SemaphoreType.DMA,
        pltpu.SemaphoreType.DMA,
        pltpu.SemaphoreType.DMA,
        pltpu.SemaphoreType.DMA,
        pltpu.SemaphoreType.DMA,
    ],
)(_gather_body)


def kernel(idx, embedding):
    idx1 = idx.reshape(-1).astype(jnp.int32)
    tabp = jnp.pad(embedding, ((0, 0), (0, _DP - _D)))
    return _gather_call(idx1, tabp)
